# Initial kernel scaffold; baseline (speedup 1.0000x reference)
import jax
import jax.numpy as jnp
from jax.experimental import pallas as pl


def kernel(x, codebook, bin_bounds):
    raise NotImplementedError("write your pallas kernel here")


# speedup vs baseline: 211.9609x; 211.9609x over previous
import jax
import jax.numpy as jnp
from jax.experimental import pallas as pl

_K = 1024
_D = 64


def _tc_body(v_ref, bounds_ref, cb_ref, o_ref):
    v = v_ref[0]
    bounds = bounds_ref[0]
    g = jnp.zeros((v.shape[0], _K), jnp.float32)
    for d in range(_D):
        g = g + (v[:, d:d + 1] > bounds).astype(jnp.float32)
    counts = jnp.concatenate(
        [jnp.float32(_D) - g[:, :1], g[:, :-1] - g[:, 1:]], axis=1)
    o_ref[0] = jnp.dot(counts, cb_ref[...],
                       preferred_element_type=jnp.float32) * (1.0 / _D)


def kernel(x, codebook, bin_bounds):
    B, S, D = x.shape
    mean = jnp.mean(x, axis=1, keepdims=True)
    scaled_x = x / (mean + 1e-06)
    min_val = jnp.min(scaled_x, axis=1, keepdims=True)
    max_val = jnp.max(scaled_x, axis=1, keepdims=True)
    normalized = (scaled_x - min_val) / (max_val - min_val + 1e-06)
    bounds_pad = jnp.concatenate(
        [bin_bounds, jnp.array([jnp.inf], jnp.float32)]).reshape(1, _K)
    return pl.pallas_call(
        _tc_body,
        grid=(B,),
        in_specs=[
            pl.BlockSpec((1, S, D), lambda b: (b, 0, 0)),
            pl.BlockSpec((1, _K), lambda b: (0, 0)),
            pl.BlockSpec((_K, D), lambda b: (0, 0)),
        ],
        out_specs=pl.BlockSpec((1, S, D), lambda b: (b, 0, 0)),
        out_shape=jax.ShapeDtypeStruct((B, S, D), jnp.float32),
    )(normalized, bounds_pad, codebook)
